# baseline (speedup 1.0000x reference)
import jax
import jax.numpy as jnp
from jax.experimental import pallas as pl
from jax.experimental.pallas import tpu as pltpu

B, C, H, W = 16, 256, 32, 32
HW = H * W
K = 1024


def _body(xb_ref, bank_ref, bank_t_ref, recon_ref, idx_ref, pn_ref):
    @pl.when(pl.program_id(0) == 0)
    def _():
        bank = bank_ref[...]
        pnorm = jnp.sqrt(jnp.sum(bank * bank, axis=1, keepdims=True))
        pn_ref[...] = bank / jnp.maximum(pnorm, 1e-12)

    xb = xb_ref[0].reshape(C, HW)
    bank_t = bank_t_ref[...]
    xnorm = jnp.sqrt(jnp.sum(xb * xb, axis=0, keepdims=True))
    xn = xb / jnp.maximum(xnorm, 1e-12)
    sims = jax.lax.dot_general(
        pn_ref[...], xn, (((1,), (0,)), ((), ())),
        preferred_element_type=jnp.float32)
    m = jnp.max(sims, axis=0, keepdims=True)
    iota = jax.lax.broadcasted_iota(jnp.int32, (K, HW), 0)
    idx = jnp.min(jnp.where(sims == m, iota, K), axis=0, keepdims=True)
    idx_ref[0] = idx
    onehot = (iota == idx).astype(jnp.float32)
    recon = jax.lax.dot_general(
        bank_t, onehot, (((1,), (0,)), ((), ())),
        preferred_element_type=jnp.float32)
    recon_ref[0] = recon.reshape(C, H, W)


def kernel(x, prototype_bank):
    bank_t = prototype_bank.T
    recon, idx = pl.pallas_call(
        _body,
        grid=(B,),
        in_specs=[
            pl.BlockSpec((1, C, H, W), lambda b: (b, 0, 0, 0)),
            pl.BlockSpec((K, C), lambda b: (0, 0)),
            pl.BlockSpec((C, K), lambda b: (0, 0)),
        ],
        out_specs=[
            pl.BlockSpec((1, C, H, W), lambda b: (b, 0, 0, 0)),
            pl.BlockSpec((1, 1, HW), lambda b: (b, 0, 0)),
        ],
        out_shape=[
            jax.ShapeDtypeStruct((B, C, H, W), jnp.float32),
            jax.ShapeDtypeStruct((B, 1, HW), jnp.int32),
        ],
        scratch_shapes=[pltpu.VMEM((K, C), jnp.float32)],
    )(x, prototype_bank, bank_t)
    return recon, idx.reshape(B, HW)


# speedup vs baseline: 1.4523x; 1.4523x over previous
import jax
import jax.numpy as jnp
from jax.experimental import pallas as pl
from jax.experimental.pallas import tpu as pltpu

B, C, H, W = 16, 256, 32, 32
HW = H * W
K = 1024


def _body(xb_ref, bank_ref, bank_t_ref, recon_ref, idx_ref, pn_ref):
    @pl.when(pl.program_id(0) == 0)
    def _():
        bank = bank_ref[...]
        pnorm = jnp.sqrt(jnp.sum(bank * bank, axis=1, keepdims=True))
        pn_ref[...] = bank / jnp.maximum(pnorm, 1e-12)

    xb = xb_ref[0]
    bank_t = bank_t_ref[...]
    xnorm = jnp.sqrt(jnp.sum(xb * xb, axis=0, keepdims=True))
    xn = xb / jnp.maximum(xnorm, 1e-12)
    sims = jax.lax.dot_general(
        pn_ref[...], xn, (((1,), (0,)), ((), ())),
        preferred_element_type=jnp.float32)
    m = jnp.max(sims, axis=0, keepdims=True)
    iota = jax.lax.broadcasted_iota(jnp.int32, (K, HW), 0)
    idx = jnp.min(jnp.where(sims == m, iota, K), axis=0, keepdims=True)
    idx_ref[0] = idx
    onehot = (iota == idx).astype(jnp.float32)
    recon = jax.lax.dot_general(
        bank_t, onehot, (((1,), (0,)), ((), ())),
        preferred_element_type=jnp.float32)
    recon_ref[0] = recon.reshape(C, H, W)


def kernel(x, prototype_bank):
    bank_t = prototype_bank.T
    xb = x.reshape(B, C, HW)
    recon, idx = pl.pallas_call(
        _body,
        grid=(B,),
        in_specs=[
            pl.BlockSpec((1, C, HW), lambda b: (b, 0, 0)),
            pl.BlockSpec((K, C), lambda b: (0, 0)),
            pl.BlockSpec((C, K), lambda b: (0, 0)),
        ],
        out_specs=[
            pl.BlockSpec((1, C, H, W), lambda b: (b, 0, 0, 0)),
            pl.BlockSpec((1, 1, HW), lambda b: (b, 0, 0)),
        ],
        out_shape=[
            jax.ShapeDtypeStruct((B, C, H, W), jnp.float32),
            jax.ShapeDtypeStruct((B, 1, HW), jnp.int32),
        ],
        scratch_shapes=[pltpu.VMEM((K, C), jnp.float32)],
    )(xb, prototype_bank, bank_t)
    return recon, idx.reshape(B, HW)


# speedup vs baseline: 2.5848x; 1.7798x over previous
import jax
import jax.numpy as jnp
from jax.experimental import pallas as pl
from jax.experimental.pallas import tpu as pltpu

B, C, H, W = 16, 256, 32, 32
HW = H * W
K = 1024


def _body(xb_ref, bank_ref, bank_t_ref, recon_ref, idx_ref, pn_ref):
    @pl.when(pl.program_id(0) == 0)
    def _():
        bank = bank_ref[...]
        pnorm = jnp.sqrt(jnp.sum(bank * bank, axis=1, keepdims=True))
        pn_ref[...] = bank / jnp.maximum(pnorm, 1e-12)

    xb = xb_ref[0]
    bank_t = bank_t_ref[...]
    xnorm = jnp.sqrt(jnp.sum(xb * xb, axis=0, keepdims=True))
    xn = xb / jnp.maximum(xnorm, 1e-12)
    sims = jax.lax.dot_general(
        pn_ref[...], xn, (((1,), (0,)), ((), ())),
        preferred_element_type=jnp.float32)
    iota = jax.lax.broadcasted_iota(jnp.int32, (K, HW), 0)
    idx = jnp.argmax(sims, axis=0)[None, :].astype(jnp.int32)
    idx_ref[0] = idx
    onehot = (iota == idx).astype(jnp.float32)
    recon = jax.lax.dot_general(
        bank_t, onehot, (((1,), (0,)), ((), ())),
        preferred_element_type=jnp.float32)
    recon_ref[0] = recon


def kernel(x, prototype_bank):
    bank_t = prototype_bank.T
    xb = x.reshape(B, C, HW)
    recon, idx = pl.pallas_call(
        _body,
        grid=(B,),
        in_specs=[
            pl.BlockSpec((1, C, HW), lambda b: (b, 0, 0)),
            pl.BlockSpec((K, C), lambda b: (0, 0)),
            pl.BlockSpec((C, K), lambda b: (0, 0)),
        ],
        out_specs=[
            pl.BlockSpec((1, C, HW), lambda b: (b, 0, 0)),
            pl.BlockSpec((1, 1, HW), lambda b: (b, 0, 0)),
        ],
        out_shape=[
            jax.ShapeDtypeStruct((B, C, HW), jnp.float32),
            jax.ShapeDtypeStruct((B, 1, HW), jnp.int32),
        ],
        scratch_shapes=[pltpu.VMEM((K, C), jnp.float32)],
    )(xb, prototype_bank, bank_t)
    return recon.reshape(B, C, H, W), idx.reshape(B, HW)


# speedup vs baseline: 2.6122x; 1.0106x over previous
import jax
import jax.numpy as jnp
from jax.experimental import pallas as pl
from jax.experimental.pallas import tpu as pltpu

B, C, H, W = 16, 256, 32, 32
HW = H * W
K = 1024


def _body(xb_ref, bank_ref, bank_t_ref, recon_ref, idx_ref, pn_ref):
    @pl.when(pl.program_id(0) == 0)
    def _():
        bank = bank_ref[...]
        pnorm = jnp.sqrt(jnp.sum(bank * bank, axis=1, keepdims=True))
        pn_ref[...] = bank / jnp.maximum(pnorm, 1e-12)

    xb = xb_ref[0]
    xnorm = jnp.sqrt(jnp.sum(xb * xb, axis=0, keepdims=True))
    xn = xb / jnp.maximum(xnorm, 1e-12)
    sims = jax.lax.dot_general(
        pn_ref[...], xn, (((1,), (0,)), ((), ())),
        preferred_element_type=jnp.float32)
    iota = jax.lax.broadcasted_iota(jnp.int32, (K, HW), 0)
    idx = jnp.argmax(sims, axis=0)[None, :].astype(jnp.int32)
    idx_ref[0] = idx
    onehot = (iota == idx).astype(jnp.bfloat16)
    recon = jax.lax.dot_general(
        bank_t_ref[...], onehot, (((1,), (0,)), ((), ())),
        preferred_element_type=jnp.float32)
    recon_ref[0] = recon


def kernel(x, prototype_bank):
    bank_t = prototype_bank.T.astype(jnp.bfloat16)
    xb = x.reshape(B, C, HW)
    recon, idx = pl.pallas_call(
        _body,
        grid=(B,),
        in_specs=[
            pl.BlockSpec((1, C, HW), lambda b: (b, 0, 0)),
            pl.BlockSpec((K, C), lambda b: (0, 0)),
            pl.BlockSpec((C, K), lambda b: (0, 0)),
        ],
        out_specs=[
            pl.BlockSpec((1, C, HW), lambda b: (b, 0, 0)),
            pl.BlockSpec((1, 1, HW), lambda b: (b, 0, 0)),
        ],
        out_shape=[
            jax.ShapeDtypeStruct((B, C, HW), jnp.float32),
            jax.ShapeDtypeStruct((B, 1, HW), jnp.int32),
        ],
        scratch_shapes=[pltpu.VMEM((K, C), jnp.float32)],
    )(xb, prototype_bank, bank_t)
    return recon.reshape(B, C, H, W), idx.reshape(B, HW)
